# baseline (speedup 1.0000x reference)
import functools

import jax
import jax.numpy as jnp
from jax import lax
from jax.experimental import pallas as pl
from jax.experimental.pallas import tpu as pltpu
from jax.experimental.pallas import tpu_sc as plsc

BATCH = 16384
VOCAB = 100000
EMB = 100
EMBP = 128
CHUNK = 128

_GDN = lax.GatherDimensionNumbers(
    offset_dims=(), collapsed_slice_dims=(0,), start_index_map=(0,))


def _lane_shuffle(vec, idx):
    return lax.gather(
        vec, idx[:, None], _GDN, slice_sizes=(1,),
        mode=lax.GatherScatterMode.PROMISE_IN_BOUNDS)


def _log_sigmoid(x):
    m = jnp.minimum(x, 0.0)
    t = jnp.exp(-jnp.abs(x))
    z = t / (2.0 + t)
    zz = z * z
    p = z * (2.0 + zz * (2.0 / 3.0 + zz * (2.0 / 5.0 + zz * (
        2.0 / 7.0 + zz * (2.0 / 9.0)))))
    return m - p


def _tc_transpose_pad(table_t):
    cb = 512

    def body(x_ref, o_ref):
        xt = jnp.transpose(x_ref[...], (1, 0))
        o_ref[...] = jnp.pad(xt, ((0, 0), (0, EMBP - EMB)))

    return pl.pallas_call(
        body,
        grid=(pl.cdiv(VOCAB, cb),),
        in_specs=[pl.BlockSpec((EMB, cb), lambda i: (0, i))],
        out_specs=pl.BlockSpec((cb, EMBP), lambda i: (i, 0)),
        out_shape=jax.ShapeDtypeStruct((VOCAB, EMBP), jnp.float32),
    )(table_t)


def _sc_fused(inp, ctx, table):
    info = plsc.get_sparse_core_info()
    nc, ns, nl = info.num_cores, info.num_subcores, info.num_lanes
    nw = nc * ns
    bpw = BATCH // nw
    nchunk = bpw // CHUNK
    mesh = plsc.VectorSubcoreMesh(core_axis_name="c", subcore_axis_name="s")

    @functools.partial(
        pl.kernel,
        mesh=mesh,
        compiler_params=pltpu.CompilerParams(use_tc_tiling_on_sc=True),
        out_type=jax.ShapeDtypeStruct((BATCH,), jnp.float32),
        scratch_types=[
            pltpu.VMEM((bpw,), jnp.int32),
            pltpu.VMEM((bpw,), jnp.int32),
            pltpu.VMEM((CHUNK, EMBP), jnp.float32),
            pltpu.VMEM((CHUNK, EMBP), jnp.float32),
            pltpu.VMEM((CHUNK, EMBP), jnp.float32),
            pltpu.VMEM((CHUNK, EMBP), jnp.float32),
            pltpu.VMEM((bpw,), jnp.float32),
            pltpu.SemaphoreType.DMA,
            pltpu.SemaphoreType.DMA,
        ],
    )
    def k(inp_hbm, ctx_hbm, table_hbm, out_hbm,
          iidx_v, cidx_v, ib0, ib1, cb0, cb1, score_v, sem0, sem1):
        wid = lax.axis_index("s") * nc + lax.axis_index("c")
        base = wid * bpw
        pltpu.sync_copy(inp_hbm.at[pl.ds(base, bpw)], iidx_v)
        pltpu.sync_copy(ctx_hbm.at[pl.ds(base, bpw)], cidx_v)
        ibufs, cbufs, sems = [ib0, ib1], [cb0, cb1], [sem0, sem1]

        lanes = lax.iota(jnp.int32, nl)
        tail_mask = lanes >= (nl - EMB % nl)
        rots = [(lanes + k) % nl for k in (8, 4, 2, 1)]

        def fire(j):
            b = j % 2
            return (
                pltpu.async_copy(
                    table_hbm.at[iidx_v.at[pl.ds(j * CHUNK, CHUNK)]],
                    ibufs[b], sems[b]),
                pltpu.async_copy(
                    table_hbm.at[cidx_v.at[pl.ds(j * CHUNK, CHUNK)]],
                    cbufs[b], sems[b]),
            )

        def row_sum(irows, crows, i):
            acc = jnp.zeros((nl,), jnp.float32)
            for o in range(0, EMB - nl, nl):
                acc = acc + irows[i, pl.ds(o, nl)] * crows[i, pl.ds(o, nl)]
            a = irows[i, pl.ds(EMB - nl, nl)]
            b = crows[i, pl.ds(EMB - nl, nl)]
            acc = acc + jnp.where(tail_mask, a * b, 0.0)
            for rot in rots:
                acc = acc + _lane_shuffle(acc, rot)
            return acc

        pend = fire(0)
        for j in range(nchunk):
            nxt = fire(j + 1) if j + 1 < nchunk else None
            pend[0].wait()
            pend[1].wait()
            pend = nxt
            irows, crows = ibufs[j % 2], cbufs[j % 2]

            def group(g, _):
                vec = jnp.zeros((nl,), jnp.float32)
                for r in range(nl):
                    vec = jnp.where(lanes == r, row_sum(irows, crows,
                                                        g * nl + r), vec)
                score_v[pl.ds(j * CHUNK + g * nl, nl)] = _log_sigmoid(vec)
                return 0

            lax.fori_loop(0, CHUNK // nl, group, 0)

        pltpu.sync_copy(score_v, out_hbm.at[pl.ds(base, bpw)])

    return k(inp, ctx, table)


def kernel(input, context, table):
    inp = input.astype(jnp.int32)
    ctx = context.astype(jnp.int32)
    tablep = _tc_transpose_pad(table.T)
    scores = _sc_fused(inp, ctx, tablep)
    return scores.reshape(BATCH, 1)


# speedup vs baseline: 2.0951x; 2.0951x over previous
import functools

import jax
import jax.numpy as jnp
from jax import lax
from jax.experimental import pallas as pl
from jax.experimental.pallas import tpu as pltpu
from jax.experimental.pallas import tpu_sc as plsc

BATCH = 16384
VOCAB = 100000
EMB = 100
CHUNK = 128

_GDN = lax.GatherDimensionNumbers(
    offset_dims=(), collapsed_slice_dims=(0,), start_index_map=(0,))


def _lane_shuffle(vec, idx):
    return lax.gather(
        vec, idx[:, None], _GDN, slice_sizes=(1,),
        mode=lax.GatherScatterMode.PROMISE_IN_BOUNDS)


def _log_sigmoid(x):
    m = jnp.minimum(x, 0.0)
    t = jnp.exp(-jnp.abs(x))
    z = t / (2.0 + t)
    zz = z * z
    p = z * (2.0 + zz * (2.0 / 3.0 + zz * (2.0 / 5.0 + zz * (
        2.0 / 7.0 + zz * (2.0 / 9.0)))))
    return m - p


def _sc_fused(inp, ctx, table):
    info = plsc.get_sparse_core_info()
    nc, ns, nl = info.num_cores, info.num_subcores, info.num_lanes
    nw = nc * ns
    bpw = BATCH // nw
    nchunk = bpw // CHUNK
    mesh = plsc.VectorSubcoreMesh(core_axis_name="c", subcore_axis_name="s")

    @functools.partial(
        pl.kernel,
        mesh=mesh,
        compiler_params=pltpu.CompilerParams(use_tc_tiling_on_sc=True),
        out_type=jax.ShapeDtypeStruct((BATCH,), jnp.float32),
        scratch_types=[
            pltpu.VMEM((bpw,), jnp.int32),
            pltpu.VMEM((bpw,), jnp.int32),
            pltpu.VMEM((CHUNK, EMB), jnp.float32),
            pltpu.VMEM((CHUNK, EMB), jnp.float32),
            pltpu.VMEM((CHUNK, EMB), jnp.float32),
            pltpu.VMEM((CHUNK, EMB), jnp.float32),
            pltpu.VMEM((bpw,), jnp.float32),
            pltpu.SemaphoreType.DMA,
            pltpu.SemaphoreType.DMA,
        ],
    )
    def k(inp_hbm, ctx_hbm, table_hbm, out_hbm,
          iidx_v, cidx_v, ib0, ib1, cb0, cb1, score_v, sem0, sem1):
        wid = lax.axis_index("s") * nc + lax.axis_index("c")
        base = wid * bpw
        pltpu.sync_copy(inp_hbm.at[pl.ds(base, bpw)], iidx_v)
        pltpu.sync_copy(ctx_hbm.at[pl.ds(base, bpw)], cidx_v)
        ibufs, cbufs, sems = [ib0, ib1], [cb0, cb1], [sem0, sem1]

        lanes = lax.iota(jnp.int32, nl)
        tail_mask = lanes >= (nl - EMB % nl)
        rots = [(lanes + k) % nl for k in (8, 4, 2, 1)]

        def fire_group(j, g):
            b = j % 2
            vi = iidx_v[pl.ds(j * CHUNK + g * nl, nl)]
            vc = cidx_v[pl.ds(j * CHUNK + g * nl, nl)]
            rbase = g * nl
            for r in range(nl):
                pltpu.async_copy(
                    table_hbm.at[vi[r]], ibufs[b].at[rbase + r], sems[b])
                pltpu.async_copy(
                    table_hbm.at[vc[r]], cbufs[b].at[rbase + r], sems[b])

        def fire(j):
            def body(g, _):
                fire_group(j, g)
                return 0

            lax.fori_loop(0, CHUNK // nl, body, 0)

        def drain(j):
            b = j % 2
            pltpu.make_async_copy(
                table_hbm.at[pl.ds(0, CHUNK)], ibufs[b], sems[b]).wait()
            pltpu.make_async_copy(
                table_hbm.at[pl.ds(0, CHUNK)], cbufs[b], sems[b]).wait()

        def row_sum(irows, crows, i):
            acc = jnp.zeros((nl,), jnp.float32)
            for o in range(0, EMB - nl, nl):
                acc = acc + irows[i, pl.ds(o, nl)] * crows[i, pl.ds(o, nl)]
            a = irows[i, pl.ds(EMB - nl, nl)]
            b = crows[i, pl.ds(EMB - nl, nl)]
            acc = acc + jnp.where(tail_mask, a * b, 0.0)
            for rot in rots:
                acc = acc + _lane_shuffle(acc, rot)
            return acc

        fire(0)
        for j in range(nchunk):
            drain(j)
            irows, crows = ibufs[j % 2], cbufs[j % 2]
            has_next = j + 1 < nchunk

            def group(g, _):
                if has_next:
                    @pl.when(g < CHUNK // nl // 2)
                    def _():
                        fire_group(j + 1, 2 * g)
                        fire_group(j + 1, 2 * g + 1)
                vec = jnp.zeros((nl,), jnp.float32)
                for r in range(nl):
                    vec = jnp.where(lanes == r, row_sum(irows, crows,
                                                        g * nl + r), vec)
                score_v[pl.ds(j * CHUNK + g * nl, nl)] = _log_sigmoid(vec)
                return 0

            lax.fori_loop(0, CHUNK // nl, group, 0)

        pltpu.sync_copy(score_v, out_hbm.at[pl.ds(base, bpw)])

    return k(inp, ctx, table)


def kernel(input, context, table):
    inp = input.astype(jnp.int32)
    ctx = context.astype(jnp.int32)
    scores = _sc_fused(inp, ctx, table)
    return scores.reshape(BATCH, 1)
